# baseline (speedup 1.0000x reference)
import jax
import jax.numpy as jnp
from jax.experimental import pallas as pl
from jax.experimental.pallas import tpu as pltpu

N = 10000
BM = 400
BK = 1024
NK = (N + BK - 1) // BK
NPAD = NK * BK


def _s0_kernel(x_ref, w0_ref, o_ref):
    o_ref[...] = jnp.dot(
        x_ref[...].astype(jnp.bfloat16), w0_ref[...].astype(jnp.bfloat16),
        preferred_element_type=jnp.float32).astype(jnp.bfloat16)


def _phase_a_kernel(adj_ref, s0_ref, b0_ref, w1_ref, s1_ref, pout_ref,
                    rhs_ref):
    i = pl.program_id(0)
    nhid = s0_ref.shape[1]

    @pl.when(i == 0)
    def _init():
        rhs_ref[:, :nhid] = s0_ref[...]
        rhs_ref[:, nhid:] = jnp.zeros_like(rhs_ref[:, nhid:])

    a = adj_ref[...].astype(jnp.bfloat16)
    r = jnp.dot(a, rhs_ref[...], preferred_element_type=jnp.float32)
    pout_ref[...] = r[:, nhid:]
    h = jnp.maximum(r[:, :nhid] + b0_ref[...], 0.0).astype(jnp.bfloat16)
    s1_i = jnp.dot(h, w1_ref[...].astype(jnp.bfloat16),
                   preferred_element_type=jnp.float32).astype(jnp.bfloat16)
    s1_ref[...] = s1_i
    rhs_ref[pl.ds(i * BM, BM), nhid:] = s1_i


def _phase_b_kernel(adj_ref, s1_ref, pout_ref, b1_ref, o_ref, acc_ref):
    i = pl.program_id(0)
    t = pl.program_id(1)
    nt = pl.num_programs(1)
    jstart = (BM * i) // BK
    j = jstart + t

    @pl.when(t == 0)
    def _init():
        acc_ref[...] = pout_ref[...]

    row = BK * j + jax.lax.broadcasted_iota(jnp.int32, (BK, 1), 0)
    s1m = jnp.where(row >= BM * i, s1_ref[...], jnp.bfloat16(0))

    @pl.when(j < NK - 1)
    def _mid():
        acc_ref[...] += jnp.dot(adj_ref[...].astype(jnp.bfloat16), s1m,
                                preferred_element_type=jnp.float32)

    @pl.when(j == NK - 1)
    def _last():
        col = BK * j + jax.lax.broadcasted_iota(jnp.int32, (1, BK), 1)
        a = jnp.where(col < N, adj_ref[...], 0.0).astype(jnp.bfloat16)
        acc_ref[...] += jnp.dot(a, s1m, preferred_element_type=jnp.float32)

    @pl.when(t == nt - 1)
    def _fin():
        z = acc_ref[...] + b1_ref[...]
        m = jnp.max(z, axis=-1, keepdims=True)
        z = z - m
        lse = jnp.log(jnp.sum(jnp.exp(z), axis=-1, keepdims=True))
        o_ref[...] = z - lse


@jax.jit
def kernel(x, adj, W0, b0, W1, b1):
    nfeat = x.shape[1]
    nhid = W0.shape[1]
    ncls = W1.shape[1]

    s0 = pl.pallas_call(
        _s0_kernel,
        grid=(5,),
        in_specs=[
            pl.BlockSpec((N // 5, nfeat), lambda i: (i, 0)),
            pl.BlockSpec((nfeat, nhid), lambda i: (0, 0)),
        ],
        out_specs=pl.BlockSpec((N // 5, nhid), lambda i: (i, 0)),
        out_shape=jax.ShapeDtypeStruct((N, nhid), jnp.bfloat16),
    )(x, W0)

    s1, pout = pl.pallas_call(
        _phase_a_kernel,
        grid=(N // BM,),
        in_specs=[
            pl.BlockSpec((BM, N), lambda i: (i, 0)),
            pl.BlockSpec((N, nhid), lambda i: (0, 0)),
            pl.BlockSpec((1, nhid), lambda i: (0, 0)),
            pl.BlockSpec((nhid, ncls), lambda i: (0, 0)),
        ],
        out_specs=[
            pl.BlockSpec((BM, ncls), lambda i: (i, 0)),
            pl.BlockSpec((BM, ncls), lambda i: (i, 0)),
        ],
        out_shape=[
            jax.ShapeDtypeStruct((N, ncls), jnp.bfloat16),
            jax.ShapeDtypeStruct((N, ncls), jnp.float32),
        ],
        scratch_shapes=[pltpu.VMEM((N, nhid + ncls), jnp.bfloat16)],
        compiler_params=pltpu.CompilerParams(
            dimension_semantics=("arbitrary",)),
    )(adj, s0, b0.reshape(1, nhid), W1)

    s1p = jnp.pad(s1, ((0, NPAD - N), (0, 0)))

    def _adj_col(i, t):
        return (i, jnp.minimum((BM * i) // BK + t, NK - 1))

    out = pl.pallas_call(
        _phase_b_kernel,
        grid=(N // BM, NK),
        in_specs=[
            pl.BlockSpec((BM, BK), _adj_col),
            pl.BlockSpec((BK, ncls), lambda i, t: (_adj_col(i, t)[1], 0)),
            pl.BlockSpec((BM, ncls), lambda i, t: (i, 0)),
            pl.BlockSpec((1, ncls), lambda i, t: (0, 0)),
        ],
        out_specs=pl.BlockSpec((BM, ncls), lambda i, t: (i, 0)),
        out_shape=jax.ShapeDtypeStruct((N, ncls), jnp.float32),
        scratch_shapes=[pltpu.VMEM((BM, ncls), jnp.float32)],
        compiler_params=pltpu.CompilerParams(
            dimension_semantics=("arbitrary", "arbitrary")),
    )(adj, s1p, pout, b1.reshape(1, ncls))

    return out


# speedup vs baseline: 1.1092x; 1.1092x over previous
import jax
import jax.numpy as jnp
from jax.experimental import pallas as pl
from jax.experimental.pallas import tpu as pltpu

N = 10000
BM = 400
BK = 1024
NK = (N + BK - 1) // BK
NPAD = NK * BK


def _s0_kernel(x_ref, w0_ref, o_ref):
    o_ref[...] = jnp.dot(
        x_ref[...].astype(jnp.bfloat16), w0_ref[...].astype(jnp.bfloat16),
        preferred_element_type=jnp.float32).astype(jnp.bfloat16)


def _phase_a_kernel(adj_ref, s0_ref, b0_ref, w1_ref, s1_ref, pout_ref,
                    rhs_ref):
    i = pl.program_id(0)
    nhid = s0_ref.shape[1]

    @pl.when(i == 0)
    def _init():
        rhs_ref[:, :nhid] = s0_ref[...]
        rhs_ref[:, nhid:] = jnp.zeros_like(rhs_ref[:, nhid:])

    a = adj_ref[...].astype(jnp.bfloat16)
    r = jnp.dot(a, rhs_ref[...], preferred_element_type=jnp.float32)
    pout_ref[...] = r[:, nhid:]
    h = jnp.maximum(r[:, :nhid] + b0_ref[...], 0.0).astype(jnp.bfloat16)
    s1_i = jnp.dot(h, w1_ref[...].astype(jnp.bfloat16),
                   preferred_element_type=jnp.float32).astype(jnp.bfloat16)
    s1_ref[...] = s1_i
    rhs_ref[pl.ds(i * BM, BM), nhid:] = s1_i


def _phase_b_kernel(ia_ref, ja_ref, adj_ref, s1_ref, pout_ref, b1_ref,
                    o_ref, acc_ref):
    s = pl.program_id(0)
    i = ia_ref[s]
    j = ja_ref[s]
    jstart = (BM * i) // BK

    @pl.when(j == jstart)
    def _init():
        acc_ref[...] = pout_ref[...]

    row = BK * j + jax.lax.broadcasted_iota(jnp.int32, (BK, 1), 0)
    s1m = jnp.where(row >= BM * i, s1_ref[...], jnp.bfloat16(0))

    @pl.when(j < NK - 1)
    def _mid():
        acc_ref[...] += jnp.dot(adj_ref[...].astype(jnp.bfloat16), s1m,
                                preferred_element_type=jnp.float32)

    @pl.when(j == NK - 1)
    def _last():
        col = BK * j + jax.lax.broadcasted_iota(jnp.int32, (1, BK), 1)
        a = jnp.where(col < N, adj_ref[...], 0.0).astype(jnp.bfloat16)
        acc = acc_ref[...] + jnp.dot(a, s1m, preferred_element_type=jnp.float32)
        z = acc + b1_ref[...]
        m = jnp.max(z, axis=-1, keepdims=True)
        z = z - m
        lse = jnp.log(jnp.sum(jnp.exp(z), axis=-1, keepdims=True))
        o_ref[...] = z - lse


@jax.jit
def kernel(x, adj, W0, b0, W1, b1):
    nfeat = x.shape[1]
    nhid = W0.shape[1]
    ncls = W1.shape[1]

    s0 = pl.pallas_call(
        _s0_kernel,
        grid=(5,),
        in_specs=[
            pl.BlockSpec((N // 5, nfeat), lambda i: (i, 0)),
            pl.BlockSpec((nfeat, nhid), lambda i: (0, 0)),
        ],
        out_specs=pl.BlockSpec((N // 5, nhid), lambda i: (i, 0)),
        out_shape=jax.ShapeDtypeStruct((N, nhid), jnp.bfloat16),
    )(x, W0)

    s1, pout = pl.pallas_call(
        _phase_a_kernel,
        grid=(N // BM,),
        in_specs=[
            pl.BlockSpec((BM, N), lambda i: (i, 0)),
            pl.BlockSpec((N, nhid), lambda i: (0, 0)),
            pl.BlockSpec((1, nhid), lambda i: (0, 0)),
            pl.BlockSpec((nhid, ncls), lambda i: (0, 0)),
        ],
        out_specs=[
            pl.BlockSpec((BM, ncls), lambda i: (i, 0)),
            pl.BlockSpec((BM, ncls), lambda i: (i, 0)),
        ],
        out_shape=[
            jax.ShapeDtypeStruct((N, ncls), jnp.bfloat16),
            jax.ShapeDtypeStruct((N, ncls), jnp.float32),
        ],
        scratch_shapes=[pltpu.VMEM((N, nhid + ncls), jnp.bfloat16)],
        compiler_params=pltpu.CompilerParams(
            dimension_semantics=("arbitrary",)),
    )(adj, s0, b0.reshape(1, nhid), W1)

    s1p = jnp.pad(s1, ((0, NPAD - N), (0, 0)))

    ia, ja = [], []
    for i in range(N // BM):
        for j in range((BM * i) // BK, NK):
            ia.append(i)
            ja.append(j)
    ia = jnp.asarray(ia, dtype=jnp.int32)
    ja = jnp.asarray(ja, dtype=jnp.int32)

    out = pl.pallas_call(
        _phase_b_kernel,
        grid_spec=pltpu.PrefetchScalarGridSpec(
            num_scalar_prefetch=2,
            grid=(len(ia),),
            in_specs=[
                pl.BlockSpec((BM, BK), lambda s, iav, jav: (iav[s], jav[s])),
                pl.BlockSpec((BK, ncls), lambda s, iav, jav: (jav[s], 0)),
                pl.BlockSpec((BM, ncls), lambda s, iav, jav: (iav[s], 0)),
                pl.BlockSpec((1, ncls), lambda s, iav, jav: (0, 0)),
            ],
            out_specs=pl.BlockSpec((BM, ncls), lambda s, iav, jav: (iav[s], 0)),
            scratch_shapes=[pltpu.VMEM((BM, ncls), jnp.float32)],
        ),
        out_shape=jax.ShapeDtypeStruct((N, ncls), jnp.float32),
        compiler_params=pltpu.CompilerParams(
            dimension_semantics=("arbitrary",)),
    )(ia, ja, adj, s1p, pout, b1.reshape(1, ncls))

    return out


# speedup vs baseline: 1.2410x; 1.1188x over previous
import jax
import jax.numpy as jnp
from jax.experimental import pallas as pl
from jax.experimental.pallas import tpu as pltpu

N = 10000
BM = 400
BK = 2048
NK = (N + BK - 1) // BK
NPAD = NK * BK


def _s0_kernel(x_ref, w0_ref, o_ref):
    o_ref[...] = jnp.dot(
        x_ref[...].astype(jnp.bfloat16), w0_ref[...].astype(jnp.bfloat16),
        preferred_element_type=jnp.float32).astype(jnp.bfloat16)


def _phase_a_kernel(adj_ref, s0_ref, b0_ref, w1_ref, s1_ref, pout_ref,
                    rhs_ref):
    i = pl.program_id(0)
    nhid = s0_ref.shape[1]

    @pl.when(i == 0)
    def _init():
        rhs_ref[:, :nhid] = s0_ref[...]
        rhs_ref[:, nhid:] = jnp.zeros_like(rhs_ref[:, nhid:])

    a = adj_ref[...].astype(jnp.bfloat16)
    r = jnp.dot(a, rhs_ref[...], preferred_element_type=jnp.float32)
    pout_ref[...] = r[:, nhid:]
    h = jnp.maximum(r[:, :nhid] + b0_ref[...], 0.0).astype(jnp.bfloat16)
    s1_i = jnp.dot(h, w1_ref[...].astype(jnp.bfloat16),
                   preferred_element_type=jnp.float32).astype(jnp.bfloat16)
    s1_ref[...] = s1_i
    rhs_ref[pl.ds(i * BM, BM), nhid:] = s1_i


def _phase_b_kernel(ia_ref, ja_ref, adj_ref, s1_ref, pout_ref, b1_ref,
                    o_ref, acc_ref):
    s = pl.program_id(0)
    i = ia_ref[s]
    j = ja_ref[s]
    jstart = (BM * i) // BK

    @pl.when(j == jstart)
    def _init():
        acc_ref[...] = pout_ref[...]

    row = BK * j + jax.lax.broadcasted_iota(jnp.int32, (BK, 1), 0)
    s1m = jnp.where(row >= BM * i, s1_ref[...], jnp.bfloat16(0))

    @pl.when(j < NK - 1)
    def _mid():
        acc_ref[...] += jnp.dot(adj_ref[...].astype(jnp.bfloat16), s1m,
                                preferred_element_type=jnp.float32)

    @pl.when(j == NK - 1)
    def _last():
        col = BK * j + jax.lax.broadcasted_iota(jnp.int32, (1, BK), 1)
        a = jnp.where(col < N, adj_ref[...], 0.0).astype(jnp.bfloat16)
        acc = acc_ref[...] + jnp.dot(a, s1m, preferred_element_type=jnp.float32)
        z = acc + b1_ref[...]
        m = jnp.max(z, axis=-1, keepdims=True)
        z = z - m
        lse = jnp.log(jnp.sum(jnp.exp(z), axis=-1, keepdims=True))
        o_ref[...] = z - lse


@jax.jit
def kernel(x, adj, W0, b0, W1, b1):
    nfeat = x.shape[1]
    nhid = W0.shape[1]
    ncls = W1.shape[1]

    s0 = pl.pallas_call(
        _s0_kernel,
        grid=(5,),
        in_specs=[
            pl.BlockSpec((N // 5, nfeat), lambda i: (i, 0)),
            pl.BlockSpec((nfeat, nhid), lambda i: (0, 0)),
        ],
        out_specs=pl.BlockSpec((N // 5, nhid), lambda i: (i, 0)),
        out_shape=jax.ShapeDtypeStruct((N, nhid), jnp.bfloat16),
    )(x, W0)

    s1, pout = pl.pallas_call(
        _phase_a_kernel,
        grid=(N // BM,),
        in_specs=[
            pl.BlockSpec((BM, N), lambda i: (i, 0)),
            pl.BlockSpec((N, nhid), lambda i: (0, 0)),
            pl.BlockSpec((1, nhid), lambda i: (0, 0)),
            pl.BlockSpec((nhid, ncls), lambda i: (0, 0)),
        ],
        out_specs=[
            pl.BlockSpec((BM, ncls), lambda i: (i, 0)),
            pl.BlockSpec((BM, ncls), lambda i: (i, 0)),
        ],
        out_shape=[
            jax.ShapeDtypeStruct((N, ncls), jnp.bfloat16),
            jax.ShapeDtypeStruct((N, ncls), jnp.float32),
        ],
        scratch_shapes=[pltpu.VMEM((N, nhid + ncls), jnp.bfloat16)],
        compiler_params=pltpu.CompilerParams(
            dimension_semantics=("arbitrary",)),
    )(adj, s0, b0.reshape(1, nhid), W1)

    s1p = jnp.pad(s1, ((0, NPAD - N), (0, 0)))

    ia, ja = [], []
    for i in range(N // BM):
        for j in range((BM * i) // BK, NK):
            ia.append(i)
            ja.append(j)
    ia = jnp.asarray(ia, dtype=jnp.int32)
    ja = jnp.asarray(ja, dtype=jnp.int32)

    out = pl.pallas_call(
        _phase_b_kernel,
        grid_spec=pltpu.PrefetchScalarGridSpec(
            num_scalar_prefetch=2,
            grid=(len(ia),),
            in_specs=[
                pl.BlockSpec((BM, BK), lambda s, iav, jav: (iav[s], jav[s])),
                pl.BlockSpec((BK, ncls), lambda s, iav, jav: (jav[s], 0)),
                pl.BlockSpec((BM, ncls), lambda s, iav, jav: (iav[s], 0)),
                pl.BlockSpec((1, ncls), lambda s, iav, jav: (0, 0)),
            ],
            out_specs=pl.BlockSpec((BM, ncls), lambda s, iav, jav: (iav[s], 0)),
            scratch_shapes=[pltpu.VMEM((BM, ncls), jnp.float32)],
        ),
        out_shape=jax.ShapeDtypeStruct((N, ncls), jnp.float32),
        compiler_params=pltpu.CompilerParams(
            dimension_semantics=("arbitrary",)),
    )(ia, ja, adj, s1p, pout, b1.reshape(1, ncls))

    return out
